# baseline (speedup 1.0000x reference)
import functools
import jax
import jax.numpy as jnp
from jax import lax
from jax.experimental import pallas as pl
from jax.experimental.pallas import tpu as pltpu
from jax.experimental.pallas import tpu_sc as plsc

SC_N = 32768
SC_K = 16384
SC_ROWS = 1024
SC_NV = SC_N // 16
SC_NW = 32
SC_RPW = SC_ROWS // SC_NW


def _key(bits):
    m = lax.shift_right_arithmetic(bits, 31)
    return bits ^ ((m ^ -1) & 0x7FFFFFFF)


def _sc_body(x_hbm, out_hbm, bufa, bufb, bufc, hist):
    lane = lax.iota(jnp.int32, 16)
    ones = jnp.ones((16,), jnp.int32)
    wid = lax.axis_index("s") * 2 + lax.axis_index("c")

    def radix_pass(src, dst, shift, transform):
        def cl(i, carry):
            hist[pl.ds(i * 16, 16)] = jnp.zeros((16,), jnp.int32)
            return carry
        lax.fori_loop(0, 256, cl, 0)

        def ha(v, carry):
            k = src[pl.ds(v * 16, 16)]
            if transform:
                k = _key(k)
            dig = lax.shift_right_logical(k, shift) & 0xFF
            idx = dig * 16 + lane
            cnt = plsc.load_gather(hist, [idx])
            plsc.store_scatter(hist, [idx], cnt + ones)
            return carry
        lax.fori_loop(0, SC_NV, ha, 0)

        def pf(b, s_carry):
            row = hist[pl.ds(b * 16, 16)]
            csum = plsc.cumsum(row)
            tot = jnp.max(csum)
            hist[pl.ds(b * 16, 16)] = (csum - row) + s_carry
            return s_carry + tot
        lax.fori_loop(0, 256, pf, jnp.int32(0))

        def pm(v, carry):
            k = src[pl.ds(v * 16, 16)]
            if transform:
                k = _key(k)
            dig = lax.shift_right_logical(k, shift) & 0xFF
            idx = dig * 16 + lane
            pos = plsc.load_gather(hist, [idx])
            plsc.store_scatter(hist, [idx], pos + 1)
            didx = ((pos & (SC_NV - 1)) << 4) | lax.shift_right_logical(pos, 11)
            plsc.store_scatter(dst, [didx], k)
            return carry
        lax.fori_loop(0, SC_NV, pm, 0)

    def row_loop(ri, carry):
        r = wid * SC_RPW + ri
        pltpu.sync_copy(x_hbm.at[r], bufa)
        radix_pass(bufa, bufb, 0, True)
        radix_pass(bufb, bufc, 8, False)
        radix_pass(bufc, bufb, 16, False)
        radix_pass(bufb, bufc, 24, False)

        def go(j, c2):
            jj = j * 16 + lane
            src_idx = ((jj & (SC_NV - 1)) << 4) | lax.shift_right_logical(jj, 11)
            k = plsc.load_gather(bufc, [src_idx])
            bufa[pl.ds(j * 16, 16)] = _key(k)
            return c2
        lax.fori_loop(0, SC_K // 16, go, 0)
        pltpu.sync_copy(bufa.at[pl.ds(0, SC_K)], out_hbm.at[r])
        return carry

    lax.fori_loop(0, SC_RPW, row_loop, 0)


@functools.partial(
    pl.kernel,
    out_type=jax.ShapeDtypeStruct((SC_ROWS, SC_K), jnp.int32),
    mesh=plsc.VectorSubcoreMesh(core_axis_name="c", subcore_axis_name="s"),
    scratch_types=[
        pltpu.VMEM((SC_N,), jnp.int32),
        pltpu.VMEM((SC_N,), jnp.int32),
        pltpu.VMEM((SC_N,), jnp.int32),
        pltpu.VMEM((4096,), jnp.int32),
    ],
    compiler_params=pltpu.CompilerParams(needs_layout_passes=False),
)
def _sc_topk(x_hbm, out_hbm, bufa, bufb, bufc, hist):
    _sc_body(x_hbm, out_hbm, bufa, bufb, bufc, hist)


@jax.jit
def kernel(x, layer_idx):
    b, ch, n = x.shape
    xb = lax.bitcast_convert_type(x.reshape(b * ch, n), jnp.int32)
    outb = _sc_topk(xb)
    out = lax.bitcast_convert_type(outb, jnp.float32).reshape(b, ch, n // 2)
    return out + jnp.zeros((), dtype=out.dtype) * layer_idx


# speedup vs baseline: 1.1078x; 1.1078x over previous
import functools
import jax
import jax.numpy as jnp
from jax import lax
from jax.experimental import pallas as pl
from jax.experimental.pallas import tpu as pltpu
from jax.experimental.pallas import tpu_sc as plsc

SC_N = 32768
SC_K = 16384
SC_ROWS = 1024
SC_NV = SC_N // 16
SC_NW = 32
SC_RPW = SC_ROWS // SC_NW


def _key(bits):
    m = lax.shift_right_arithmetic(bits, 31)
    return bits ^ ((m ^ -1) & 0x7FFFFFFF)


def _sc_body(x_hbm, out_hbm, bufa, bufb, bufc, hist):
    lane = lax.iota(jnp.int32, 16)
    ones = jnp.ones((16,), jnp.int32)
    wid = lax.axis_index("s") * 2 + lax.axis_index("c")

    def radix_pass(src, dst, shift, transform):
        def cl(i, carry):
            hist[pl.ds(i * 16, 16)] = jnp.zeros((16,), jnp.int32)
            return carry
        lax.fori_loop(0, 256, cl, 0, unroll=8)

        def ha(v, carry):
            k = src[pl.ds(v * 16, 16)]
            if transform:
                k = _key(k)
            dig = lax.shift_right_logical(k, shift) & 0xFF
            idx = dig * 16 + lane
            cnt = plsc.load_gather(hist, [idx])
            plsc.store_scatter(hist, [idx], cnt + ones)
            return carry
        lax.fori_loop(0, SC_NV, ha, 0, unroll=8)

        def pf(b, s_carry):
            row = hist[pl.ds(b * 16, 16)]
            csum = plsc.cumsum(row)
            tot = jnp.max(csum)
            hist[pl.ds(b * 16, 16)] = (csum - row) + s_carry
            return s_carry + tot
        lax.fori_loop(0, 256, pf, jnp.int32(0), unroll=4)

        def pm(v, carry):
            k = src[pl.ds(v * 16, 16)]
            if transform:
                k = _key(k)
            dig = lax.shift_right_logical(k, shift) & 0xFF
            idx = dig * 16 + lane
            pos = plsc.load_gather(hist, [idx])
            plsc.store_scatter(hist, [idx], pos + 1)
            didx = ((pos & (SC_NV - 1)) << 4) | lax.shift_right_logical(pos, 11)
            plsc.store_scatter(dst, [didx], k)
            return carry
        lax.fori_loop(0, SC_NV, pm, 0, unroll=8)

    def row_loop(ri, carry):
        r = wid * SC_RPW + ri
        pltpu.sync_copy(x_hbm.at[r], bufa)
        radix_pass(bufa, bufb, 0, True)
        radix_pass(bufb, bufc, 8, False)
        radix_pass(bufc, bufb, 16, False)
        radix_pass(bufb, bufc, 24, False)

        def go(j, c2):
            jj = j * 16 + lane
            src_idx = ((jj & (SC_NV - 1)) << 4) | lax.shift_right_logical(jj, 11)
            k = plsc.load_gather(bufc, [src_idx])
            bufa[pl.ds(j * 16, 16)] = _key(k)
            return c2
        lax.fori_loop(0, SC_K // 16, go, 0, unroll=8)
        pltpu.sync_copy(bufa.at[pl.ds(0, SC_K)], out_hbm.at[r])
        return carry

    lax.fori_loop(0, SC_RPW, row_loop, 0)


@functools.partial(
    pl.kernel,
    out_type=jax.ShapeDtypeStruct((SC_ROWS, SC_K), jnp.int32),
    mesh=plsc.VectorSubcoreMesh(core_axis_name="c", subcore_axis_name="s"),
    scratch_types=[
        pltpu.VMEM((SC_N,), jnp.int32),
        pltpu.VMEM((SC_N,), jnp.int32),
        pltpu.VMEM((SC_N,), jnp.int32),
        pltpu.VMEM((4096,), jnp.int32),
    ],
    compiler_params=pltpu.CompilerParams(needs_layout_passes=False),
)
def _sc_topk(x_hbm, out_hbm, bufa, bufb, bufc, hist):
    _sc_body(x_hbm, out_hbm, bufa, bufb, bufc, hist)


@jax.jit
def kernel(x, layer_idx):
    b, ch, n = x.shape
    xb = lax.bitcast_convert_type(x.reshape(b * ch, n), jnp.int32)
    outb = _sc_topk(xb)
    out = lax.bitcast_convert_type(outb, jnp.float32).reshape(b, ch, n // 2)
    return out + jnp.zeros((), dtype=out.dtype) * layer_idx
